# baseline (speedup 1.0000x reference)
import jax
import jax.numpy as jnp
from jax import lax
from jax.experimental import pallas as pl
from jax.experimental.pallas import tpu as pltpu
from jax.experimental.pallas import tpu_sc as plsc

B = 16384
D = 128
L = 16
NC = 2
NS = 16
NW = NC * NS
BPW = B // NW
CH = 128
NCH = BPW // CH
DL = D // L


def _lane_sum(v, lane):
    for k in (1, 2, 4, 8):
        v = v + v.at[lane ^ k].get(mode="promise_in_bounds")
    return v


def _rinorm(acc, lane):
    s = jnp.maximum(_lane_sum(acc, lane), jnp.float32(1e-24))
    i = lax.bitcast_convert_type(s, jnp.int32)
    y = lax.bitcast_convert_type(
        jnp.int32(0x5F3759DF) - lax.shift_right_arithmetic(i, 1), jnp.float32)
    sh = jnp.float32(0.5) * s
    y = y * (jnp.float32(1.5) - sh * y * y)
    return y


def _score(hb, rb, tb, jj, lane):
    hs = [hb[jj, pl.ds(L * i, L)] for i in range(DL)]
    rs = [rb[jj, pl.ds(L * i, L)] for i in range(DL)]
    ts = [tb[jj, pl.ds(L * i, L)] for i in range(DL)]
    ah = ar = at = None
    for i in range(DL):
        ah = hs[i] * hs[i] if ah is None else ah + hs[i] * hs[i]
        ar = rs[i] * rs[i] if ar is None else ar + rs[i] * rs[i]
        at = ts[i] * ts[i] if at is None else at + ts[i] * ts[i]
    ch = _rinorm(ah, lane)
    cr = _rinorm(ar, lane)
    ct = _rinorm(at, lane)
    acc = None
    for i in range(DL):
        v = jnp.abs(hs[i] * ch + rs[i] * cr - ts[i] * ct)
        acc = v if acc is None else acc + v
    return _lane_sum(acc, lane)


def _body(hidx, ridx, tidx, ent, rel, out, hi_v, ri_v, ti_v,
          hb0, rb0, tb0, hb1, rb1, tb1, sc_v, sem0, sem1):
    wid = lax.axis_index("s") * NC + lax.axis_index("c")
    base = pl.multiple_of(wid * BPW, BPW)
    pltpu.sync_copy(hidx.at[pl.ds(base, BPW)], hi_v)
    pltpu.sync_copy(ridx.at[pl.ds(base, BPW)], ri_v)
    pltpu.sync_copy(tidx.at[pl.ds(base, BPW)], ti_v)

    lane = lax.iota(jnp.int32, L)
    bufs = ((hb0, rb0, tb0), (hb1, rb1, tb1))
    sems = (sem0, sem1)

    def issue(cidx, b):
        c = pl.multiple_of(cidx * CH, CH)
        h, r, t = bufs[b]
        pltpu.async_copy(ent.at[hi_v.at[pl.ds(c, CH)]], h, sems[b])
        pltpu.async_copy(rel.at[ri_v.at[pl.ds(c, CH)]], r, sems[b])
        pltpu.async_copy(ent.at[ti_v.at[pl.ds(c, CH)]], t, sems[b])

    def drain(b):
        h, r, t = bufs[b]
        pltpu.make_async_copy(ent.at[pl.ds(0, CH)], h, sems[b]).wait()
        pltpu.make_async_copy(rel.at[pl.ds(0, CH)], r, sems[b]).wait()
        pltpu.make_async_copy(ent.at[pl.ds(0, CH)], t, sems[b]).wait()

    def compute(cidx, b):
        h, r, t = bufs[b]
        c = pl.multiple_of(cidx * CH, CH)

        @pl.loop(0, CH // L)
        def _group(g):
            @pl.loop(0, L, step=4, init_carry=jnp.zeros((L,), jnp.float32))
            def vec(j16, acc):
                for u in range(4):
                    s = _score(h, r, t, g * L + j16 + u, lane)
                    acc = jnp.where(lane == j16 + u, s, acc)
                return acc

            off = pl.multiple_of(c + g * L, L)
            sc_v[pl.ds(off, L)] = vec

    issue(0, 0)

    @pl.loop(0, NCH, step=2)
    def _pair(ci):
        for b in range(2):
            cur = ci + b
            issue(jnp.minimum(cur + 1, NCH - 1), 1 - b)
            drain(b)
            compute(cur, b)

    drain(0)

    pltpu.sync_copy(sc_v, out.at[pl.ds(base, BPW)])


_mesh = plsc.VectorSubcoreMesh(
    core_axis_name="c", subcore_axis_name="s", num_cores=NC, num_subcores=NS)

_call = pl.kernel(
    _body,
    out_type=jax.ShapeDtypeStruct((B,), jnp.float32),
    mesh=_mesh,
    scratch_types=[
        pltpu.VMEM((BPW,), jnp.int32),
        pltpu.VMEM((BPW,), jnp.int32),
        pltpu.VMEM((BPW,), jnp.int32),
        pltpu.VMEM((CH, D), jnp.float32),
        pltpu.VMEM((CH, D), jnp.float32),
        pltpu.VMEM((CH, D), jnp.float32),
        pltpu.VMEM((CH, D), jnp.float32),
        pltpu.VMEM((CH, D), jnp.float32),
        pltpu.VMEM((CH, D), jnp.float32),
        pltpu.VMEM((BPW,), jnp.float32),
        pltpu.SemaphoreType.DMA,
        pltpu.SemaphoreType.DMA,
    ],
)


@jax.jit
def kernel(triples, ent_emb, rel_emb):
    hidx = triples[:, 0]
    ridx = triples[:, 1]
    tidx = triples[:, 2]
    return _call(hidx, ridx, tidx, ent_emb, rel_emb)


# speedup vs baseline: 1.1315x; 1.1315x over previous
import jax
import jax.numpy as jnp
from jax import lax
from jax.experimental import pallas as pl
from jax.experimental.pallas import tpu as pltpu
from jax.experimental.pallas import tpu_sc as plsc

B = 16384
D = 128
L = 16
NC = 2
NS = 16
NW = NC * NS
BPW = B // NW
CH = 128
NCH = BPW // CH
DL = D // L


def _lane_sum(v, lane):
    for k in (1, 2, 4, 8):
        v = v + v.at[lane ^ k].get(mode="promise_in_bounds")
    return v


def _rinorm(acc, lane):
    s = jnp.maximum(_lane_sum(acc, lane), jnp.float32(1e-24))
    i = lax.bitcast_convert_type(s, jnp.int32)
    y = lax.bitcast_convert_type(
        jnp.int32(0x5F3759DF) - lax.shift_right_arithmetic(i, 1), jnp.float32)
    sh = jnp.float32(0.5) * s
    y = y * (jnp.float32(1.5) - sh * y * y)
    return y


def _score(hb, rb, tb, jj, lane):
    hs = [hb[jj, pl.ds(L * i, L)] for i in range(DL)]
    rs = [rb[jj, pl.ds(L * i, L)] for i in range(DL)]
    ts = [tb[jj, pl.ds(L * i, L)] for i in range(DL)]
    ah = ar = at = None
    for i in range(DL):
        ah = hs[i] * hs[i] if ah is None else ah + hs[i] * hs[i]
        ar = rs[i] * rs[i] if ar is None else ar + rs[i] * rs[i]
        at = ts[i] * ts[i] if at is None else at + ts[i] * ts[i]
    ch = _rinorm(ah, lane)
    cr = _rinorm(ar, lane)
    ct = _rinorm(at, lane)
    acc = None
    for i in range(DL):
        v = jnp.abs(hs[i] * ch + rs[i] * cr - ts[i] * ct)
        acc = v if acc is None else acc + v
    return _lane_sum(acc, lane)


def _body(hidx, ridx, tidx, ent, rel, out, hi_v, ri_v, ti_v,
          hb0, rb0, tb0, hb1, rb1, tb1, sc_v, sem0, sem1):
    wid = lax.axis_index("s") * NC + lax.axis_index("c")
    base = pl.multiple_of(wid * BPW, BPW)
    pltpu.sync_copy(hidx.at[pl.ds(base, BPW)], hi_v)
    pltpu.sync_copy(ridx.at[pl.ds(base, BPW)], ri_v)
    pltpu.sync_copy(tidx.at[pl.ds(base, BPW)], ti_v)

    lane = lax.iota(jnp.int32, L)
    bufs = ((hb0, rb0, tb0), (hb1, rb1, tb1))
    sems = (sem0, sem1)

    def issue(cidx, b):
        c = pl.multiple_of(cidx * CH, CH)
        h, r, t = bufs[b]
        pltpu.async_copy(ent.at[hi_v.at[pl.ds(c, CH)]], h, sems[b])
        pltpu.async_copy(rel.at[ri_v.at[pl.ds(c, CH)]], r, sems[b])
        pltpu.async_copy(ent.at[ti_v.at[pl.ds(c, CH)]], t, sems[b])

    def drain(b):
        h, r, t = bufs[b]
        pltpu.make_async_copy(ent.at[pl.ds(0, CH)], h, sems[b]).wait()
        pltpu.make_async_copy(rel.at[pl.ds(0, CH)], r, sems[b]).wait()
        pltpu.make_async_copy(ent.at[pl.ds(0, CH)], t, sems[b]).wait()

    def compute(cidx, b):
        h, r, t = bufs[b]
        c = pl.multiple_of(cidx * CH, CH)

        @pl.loop(0, CH // L)
        def _group(g):
            @pl.loop(0, L, step=2, init_carry=jnp.zeros((L,), jnp.float32))
            def vec(j16, acc):
                s0 = _score(h, r, t, g * L + j16, lane)
                s1 = _score(h, r, t, g * L + j16 + 1, lane)
                acc = jnp.where(lane == j16, s0, acc)
                return jnp.where(lane == j16 + 1, s1, acc)

            off = pl.multiple_of(c + g * L, L)
            sc_v[pl.ds(off, L)] = vec

    issue(0, 0)

    @pl.loop(0, NCH, step=2)
    def _pair(ci):
        for b in range(2):
            cur = ci + b
            issue(jnp.minimum(cur + 1, NCH - 1), 1 - b)
            drain(b)
            compute(cur, b)

    drain(0)

    pltpu.sync_copy(sc_v, out.at[pl.ds(base, BPW)])


_mesh = plsc.VectorSubcoreMesh(
    core_axis_name="c", subcore_axis_name="s", num_cores=NC, num_subcores=NS)

_call = pl.kernel(
    _body,
    out_type=jax.ShapeDtypeStruct((B,), jnp.float32),
    mesh=_mesh,
    scratch_types=[
        pltpu.VMEM((BPW,), jnp.int32),
        pltpu.VMEM((BPW,), jnp.int32),
        pltpu.VMEM((BPW,), jnp.int32),
        pltpu.VMEM((CH, D), jnp.float32),
        pltpu.VMEM((CH, D), jnp.float32),
        pltpu.VMEM((CH, D), jnp.float32),
        pltpu.VMEM((CH, D), jnp.float32),
        pltpu.VMEM((CH, D), jnp.float32),
        pltpu.VMEM((CH, D), jnp.float32),
        pltpu.VMEM((BPW,), jnp.float32),
        pltpu.SemaphoreType.DMA,
        pltpu.SemaphoreType.DMA,
    ],
)


@jax.jit
def kernel(triples, ent_emb, rel_emb):
    hidx = triples[:, 0]
    ridx = triples[:, 1]
    tidx = triples[:, 2]
    return _call(hidx, ridx, tidx, ent_emb, rel_emb)


# speedup vs baseline: 1.1868x; 1.0488x over previous
import jax
import jax.numpy as jnp
from jax import lax
from jax.experimental import pallas as pl
from jax.experimental.pallas import tpu as pltpu
from jax.experimental.pallas import tpu_sc as plsc

B = 16384
D = 128
L = 16
NC = 2
NS = 16
NW = NC * NS
BPW = B // NW
CH = 64
NCH = BPW // CH
DL = D // L


def _lane_sum(v, lane):
    for k in (1, 2, 4, 8):
        v = v + v.at[lane ^ k].get(mode="promise_in_bounds")
    return v


def _rinorm(acc, lane):
    s = jnp.maximum(_lane_sum(acc, lane), jnp.float32(1e-24))
    i = lax.bitcast_convert_type(s, jnp.int32)
    y = lax.bitcast_convert_type(
        jnp.int32(0x5F3759DF) - lax.shift_right_arithmetic(i, 1), jnp.float32)
    sh = jnp.float32(0.5) * s
    y = y * (jnp.float32(1.5) - sh * y * y)
    return y


def _score(hb, rb, tb, jj, lane):
    hs = [hb[jj, pl.ds(L * i, L)] for i in range(DL)]
    rs = [rb[jj, pl.ds(L * i, L)] for i in range(DL)]
    ts = [tb[jj, pl.ds(L * i, L)] for i in range(DL)]
    ah = ar = at = None
    for i in range(DL):
        ah = hs[i] * hs[i] if ah is None else ah + hs[i] * hs[i]
        ar = rs[i] * rs[i] if ar is None else ar + rs[i] * rs[i]
        at = ts[i] * ts[i] if at is None else at + ts[i] * ts[i]
    ch = _rinorm(ah, lane)
    cr = _rinorm(ar, lane)
    ct = _rinorm(at, lane)
    acc = None
    for i in range(DL):
        v = jnp.abs(hs[i] * ch + rs[i] * cr - ts[i] * ct)
        acc = v if acc is None else acc + v
    return _lane_sum(acc, lane)


def _body(hidx, ridx, tidx, ent, rel, out, hi_v, ri_v, ti_v,
          hb0, rb0, tb0, hb1, rb1, tb1, sc_v, sem0, sem1):
    wid = lax.axis_index("s") * NC + lax.axis_index("c")
    base = pl.multiple_of(wid * BPW, BPW)
    pltpu.sync_copy(hidx.at[pl.ds(base, BPW)], hi_v)
    pltpu.sync_copy(ridx.at[pl.ds(base, BPW)], ri_v)
    pltpu.sync_copy(tidx.at[pl.ds(base, BPW)], ti_v)

    lane = lax.iota(jnp.int32, L)
    bufs = ((hb0, rb0, tb0), (hb1, rb1, tb1))
    sems = (sem0, sem1)

    def issue(cidx, b):
        c = pl.multiple_of(cidx * CH, CH)
        h, r, t = bufs[b]
        pltpu.async_copy(ent.at[hi_v.at[pl.ds(c, CH)]], h, sems[b])
        pltpu.async_copy(rel.at[ri_v.at[pl.ds(c, CH)]], r, sems[b])
        pltpu.async_copy(ent.at[ti_v.at[pl.ds(c, CH)]], t, sems[b])

    def drain(b):
        h, r, t = bufs[b]
        pltpu.make_async_copy(ent.at[pl.ds(0, CH)], h, sems[b]).wait()
        pltpu.make_async_copy(rel.at[pl.ds(0, CH)], r, sems[b]).wait()
        pltpu.make_async_copy(ent.at[pl.ds(0, CH)], t, sems[b]).wait()

    def compute(cidx, b):
        h, r, t = bufs[b]
        c = pl.multiple_of(cidx * CH, CH)

        @pl.loop(0, CH // L)
        def _group(g):
            @pl.loop(0, L, step=2, init_carry=jnp.zeros((L,), jnp.float32))
            def vec(j16, acc):
                s0 = _score(h, r, t, g * L + j16, lane)
                s1 = _score(h, r, t, g * L + j16 + 1, lane)
                acc = jnp.where(lane == j16, s0, acc)
                return jnp.where(lane == j16 + 1, s1, acc)

            off = pl.multiple_of(c + g * L, L)
            sc_v[pl.ds(off, L)] = vec

    issue(0, 0)

    @pl.loop(0, NCH, step=2)
    def _pair(ci):
        for b in range(2):
            cur = ci + b
            issue(jnp.minimum(cur + 1, NCH - 1), 1 - b)
            drain(b)
            compute(cur, b)

    drain(0)

    pltpu.sync_copy(sc_v, out.at[pl.ds(base, BPW)])


_mesh = plsc.VectorSubcoreMesh(
    core_axis_name="c", subcore_axis_name="s", num_cores=NC, num_subcores=NS)

_call = pl.kernel(
    _body,
    out_type=jax.ShapeDtypeStruct((B,), jnp.float32),
    mesh=_mesh,
    scratch_types=[
        pltpu.VMEM((BPW,), jnp.int32),
        pltpu.VMEM((BPW,), jnp.int32),
        pltpu.VMEM((BPW,), jnp.int32),
        pltpu.VMEM((CH, D), jnp.float32),
        pltpu.VMEM((CH, D), jnp.float32),
        pltpu.VMEM((CH, D), jnp.float32),
        pltpu.VMEM((CH, D), jnp.float32),
        pltpu.VMEM((CH, D), jnp.float32),
        pltpu.VMEM((CH, D), jnp.float32),
        pltpu.VMEM((BPW,), jnp.float32),
        pltpu.SemaphoreType.DMA,
        pltpu.SemaphoreType.DMA,
    ],
)


@jax.jit
def kernel(triples, ent_emb, rel_emb):
    hidx = triples[:, 0]
    ridx = triples[:, 1]
    tidx = triples[:, 2]
    return _call(hidx, ridx, tidx, ent_emb, rel_emb)


# speedup vs baseline: 1.1907x; 1.0033x over previous
import jax
import jax.numpy as jnp
from jax import lax
from jax.experimental import pallas as pl
from jax.experimental.pallas import tpu as pltpu
from jax.experimental.pallas import tpu_sc as plsc

B = 16384
D = 128
L = 16
NC = 2
NS = 16
NW = NC * NS
BPW = B // NW
CH = 32
NCH = BPW // CH
DL = D // L


def _lane_sum(v, lane):
    for k in (1, 2, 4, 8):
        v = v + v.at[lane ^ k].get(mode="promise_in_bounds")
    return v


def _rinorm(acc, lane):
    s = jnp.maximum(_lane_sum(acc, lane), jnp.float32(1e-24))
    i = lax.bitcast_convert_type(s, jnp.int32)
    y = lax.bitcast_convert_type(
        jnp.int32(0x5F3759DF) - lax.shift_right_arithmetic(i, 1), jnp.float32)
    sh = jnp.float32(0.5) * s
    y = y * (jnp.float32(1.5) - sh * y * y)
    return y


def _score(hb, rb, tb, jj, lane):
    hs = [hb[jj, pl.ds(L * i, L)] for i in range(DL)]
    rs = [rb[jj, pl.ds(L * i, L)] for i in range(DL)]
    ts = [tb[jj, pl.ds(L * i, L)] for i in range(DL)]
    ah = ar = at = None
    for i in range(DL):
        ah = hs[i] * hs[i] if ah is None else ah + hs[i] * hs[i]
        ar = rs[i] * rs[i] if ar is None else ar + rs[i] * rs[i]
        at = ts[i] * ts[i] if at is None else at + ts[i] * ts[i]
    ch = _rinorm(ah, lane)
    cr = _rinorm(ar, lane)
    ct = _rinorm(at, lane)
    acc = None
    for i in range(DL):
        v = jnp.abs(hs[i] * ch + rs[i] * cr - ts[i] * ct)
        acc = v if acc is None else acc + v
    return _lane_sum(acc, lane)


def _body(hidx, ridx, tidx, ent, rel, out, hi_v, ri_v, ti_v,
          hb0, rb0, tb0, hb1, rb1, tb1, sc_v, sem0, sem1):
    wid = lax.axis_index("s") * NC + lax.axis_index("c")
    base = pl.multiple_of(wid * BPW, BPW)
    pltpu.sync_copy(hidx.at[pl.ds(base, BPW)], hi_v)
    pltpu.sync_copy(ridx.at[pl.ds(base, BPW)], ri_v)
    pltpu.sync_copy(tidx.at[pl.ds(base, BPW)], ti_v)

    lane = lax.iota(jnp.int32, L)
    bufs = ((hb0, rb0, tb0), (hb1, rb1, tb1))
    sems = (sem0, sem1)

    def issue(cidx, b):
        c = pl.multiple_of(cidx * CH, CH)
        h, r, t = bufs[b]
        pltpu.async_copy(ent.at[hi_v.at[pl.ds(c, CH)]], h, sems[b])
        pltpu.async_copy(rel.at[ri_v.at[pl.ds(c, CH)]], r, sems[b])
        pltpu.async_copy(ent.at[ti_v.at[pl.ds(c, CH)]], t, sems[b])

    def drain(b):
        h, r, t = bufs[b]
        pltpu.make_async_copy(ent.at[pl.ds(0, CH)], h, sems[b]).wait()
        pltpu.make_async_copy(rel.at[pl.ds(0, CH)], r, sems[b]).wait()
        pltpu.make_async_copy(ent.at[pl.ds(0, CH)], t, sems[b]).wait()

    def compute(cidx, b):
        h, r, t = bufs[b]
        c = pl.multiple_of(cidx * CH, CH)

        @pl.loop(0, CH // L)
        def _group(g):
            @pl.loop(0, L, step=2, init_carry=jnp.zeros((L,), jnp.float32))
            def vec(j16, acc):
                s0 = _score(h, r, t, g * L + j16, lane)
                s1 = _score(h, r, t, g * L + j16 + 1, lane)
                acc = jnp.where(lane == j16, s0, acc)
                return jnp.where(lane == j16 + 1, s1, acc)

            off = pl.multiple_of(c + g * L, L)
            sc_v[pl.ds(off, L)] = vec

    issue(0, 0)

    @pl.loop(0, NCH, step=2)
    def _pair(ci):
        for b in range(2):
            cur = ci + b
            issue(jnp.minimum(cur + 1, NCH - 1), 1 - b)
            drain(b)
            compute(cur, b)

    drain(0)

    pltpu.sync_copy(sc_v, out.at[pl.ds(base, BPW)])


_mesh = plsc.VectorSubcoreMesh(
    core_axis_name="c", subcore_axis_name="s", num_cores=NC, num_subcores=NS)

_call = pl.kernel(
    _body,
    out_type=jax.ShapeDtypeStruct((B,), jnp.float32),
    mesh=_mesh,
    scratch_types=[
        pltpu.VMEM((BPW,), jnp.int32),
        pltpu.VMEM((BPW,), jnp.int32),
        pltpu.VMEM((BPW,), jnp.int32),
        pltpu.VMEM((CH, D), jnp.float32),
        pltpu.VMEM((CH, D), jnp.float32),
        pltpu.VMEM((CH, D), jnp.float32),
        pltpu.VMEM((CH, D), jnp.float32),
        pltpu.VMEM((CH, D), jnp.float32),
        pltpu.VMEM((CH, D), jnp.float32),
        pltpu.VMEM((BPW,), jnp.float32),
        pltpu.SemaphoreType.DMA,
        pltpu.SemaphoreType.DMA,
    ],
)


@jax.jit
def kernel(triples, ent_emb, rel_emb):
    hidx = triples[:, 0]
    ridx = triples[:, 1]
    tidx = triples[:, 2]
    return _call(hidx, ridx, tidx, ent_emb, rel_emb)
